# baseline (speedup 1.0000x reference)
import functools

import jax
import jax.numpy as jnp
from jax import lax
from jax.experimental import pallas as pl
from jax.experimental.pallas import tpu as pltpu
from jax.experimental.pallas import tpu_sc as plsc

E = 8
TOP_K = 2
D_IN = 2048
D_HID = 4096
D_OUT = 2048
B = 4096

EP = 128
BT = 256
NBG = B // BT
NEG = -1e30

T = 256
NROWS = 2 * B + E * T
NT = NROWS // T
NE = 2 * B
EBLK = 512
NEB = NE // EBLK

HB = 2
DH = D_HID // HB


def _gating_body(x_ref, wg_ref, bg_ref, e1_ref, e2_ref, w0_ref, w1_ref,
                 loss_ref, usage_ref):
    jb = pl.program_id(0)
    logits = jax.lax.dot(x_ref[...], wg_ref[...],
                         precision=jax.lax.Precision.DEFAULT) + bg_ref[...]
    iota = jax.lax.broadcasted_iota(jnp.int32, (BT, EP), 1)
    big = jnp.int32(2**30)
    m1 = jnp.max(logits, axis=1, keepdims=True)
    i1 = jnp.min(jnp.where(logits == m1, iota, big), axis=1, keepdims=True)
    l2 = jnp.where(iota == i1, NEG, logits)
    m2 = jnp.max(l2, axis=1, keepdims=True)
    i2 = jnp.min(jnp.where(l2 == m2, iota, big), axis=1, keepdims=True)
    d = jnp.exp(m2 - m1)
    w0 = 1.0 / (1.0 + d)
    w1 = d / (1.0 + d)
    e1_ref[...] = i1.reshape(1, BT, 1)
    e2_ref[...] = i2.reshape(1, BT, 1)
    w0_ref[...] = w0.reshape(1, BT, 1)
    w1_ref[...] = w1.reshape(1, BT, 1)
    g = jnp.exp(logits - m1)
    g = g / jnp.sum(g, axis=1, keepdims=True)
    gsum = jnp.sum(g, axis=0, keepdims=True)

    @pl.when(jb == 0)
    def _():
        usage_ref[...] = jnp.zeros_like(usage_ref)

    usage_ref[...] += gsum

    @pl.when(jb == NBG - 1)
    def _():
        usage = usage_ref[...] / B
        loss_ref[...] = jnp.reshape(E * jnp.sum(usage * usage), (1, 1))


def _gating(x, wgp, bgp):
    return pl.pallas_call(
        _gating_body,
        grid=(NBG,),
        in_specs=[
            pl.BlockSpec((BT, D_IN), lambda jb: (jb, 0)),
            pl.BlockSpec((D_IN, EP), lambda jb: (0, 0)),
            pl.BlockSpec((1, EP), lambda jb: (0, 0)),
        ],
        out_specs=[
            pl.BlockSpec((1, BT, 1), lambda jb: (jb, 0, 0)),
            pl.BlockSpec((1, BT, 1), lambda jb: (jb, 0, 0)),
            pl.BlockSpec((1, BT, 1), lambda jb: (jb, 0, 0)),
            pl.BlockSpec((1, BT, 1), lambda jb: (jb, 0, 0)),
            pl.BlockSpec((1, 1), lambda jb: (0, 0)),
        ],
        out_shape=[
            jax.ShapeDtypeStruct((NBG, BT, 1), jnp.int32),
            jax.ShapeDtypeStruct((NBG, BT, 1), jnp.int32),
            jax.ShapeDtypeStruct((NBG, BT, 1), jnp.float32),
            jax.ShapeDtypeStruct((NBG, BT, 1), jnp.float32),
            jax.ShapeDtypeStruct((1, 1), jnp.float32),
        ],
        scratch_shapes=[pltpu.VMEM((1, EP), jnp.float32)],
    )(x, wgp, bgp)


def _meta1_body(eids_ref, ranks_ref, offs_ref, tmeta_ref, carry_ref):
    s = pl.program_id(0)
    eids = eids_ref[0]
    lane = jax.lax.broadcasted_iota(jnp.int32, (EBLK, EP), 1)
    onehot = (eids == lane).astype(jnp.float32)
    r0 = jax.lax.broadcasted_iota(jnp.int32, (EBLK, EBLK), 0)
    r1 = jax.lax.broadcasted_iota(jnp.int32, (EBLK, EBLK), 1)
    lstrict = (r0 > r1).astype(jnp.float32)
    run = jax.lax.dot(lstrict, onehot,
                      precision=jax.lax.Precision.DEFAULT)

    @pl.when(s == 0)
    def _():
        carry_ref[...] = jnp.zeros_like(carry_ref)

    carry = carry_ref[...]
    rank = jnp.sum((run + carry) * onehot, axis=1, keepdims=True)
    ranks_ref[...] = rank.astype(jnp.int32).reshape(1, EBLK, 1)
    carry_ref[...] = carry + jnp.sum(onehot, axis=0, keepdims=True)

    @pl.when(s == NEB - 1)
    def _():
        counts = carry_ref[...]
        padded = jnp.ceil(counts / T) * T
        up0 = jax.lax.broadcasted_iota(jnp.int32, (EP, EP), 0)
        up1 = jax.lax.broadcasted_iota(jnp.int32, (EP, EP), 1)
        umask = (up0 < up1).astype(jnp.float32)
        offs = jax.lax.dot(padded, umask,
                           precision=jax.lax.Precision.HIGHEST)
        offs_ref[...] = offs.astype(jnp.int32)
        ends = (offs + padded) / T
        jt0 = jax.lax.broadcasted_iota(jnp.int32, (EP, EP), 0).astype(jnp.float32)
        laner = jax.lax.broadcasted_iota(jnp.int32, (EP, EP), 1)
        ge = jnp.where((laner < E) & (jt0 >= jnp.broadcast_to(ends, (EP, EP))),
                       1.0, 0.0)
        te = jnp.sum(ge, axis=1, keepdims=True)
        te = jnp.minimum(te, E - 1.0)
        nact = jnp.sum(padded) / T
        col = jax.lax.broadcasted_iota(jnp.int32, (EP, 1), 0)
        tmeta_ref[...] = jnp.where(col == EP - 1, nact, te).astype(jnp.int32)


def _meta1(eids3):
    return pl.pallas_call(
        _meta1_body,
        grid=(NEB,),
        in_specs=[pl.BlockSpec((1, EBLK, 1), lambda s: (s, 0, 0))],
        out_specs=[
            pl.BlockSpec((1, EBLK, 1), lambda s: (s, 0, 0)),
            pl.BlockSpec((1, EP), lambda s: (0, 0)),
            pl.BlockSpec((EP, 1), lambda s: (0, 0)),
        ],
        out_shape=[
            jax.ShapeDtypeStruct((NEB, EBLK, 1), jnp.int32),
            jax.ShapeDtypeStruct((1, EP), jnp.int32),
            jax.ShapeDtypeStruct((EP, 1), jnp.int32),
        ],
        scratch_shapes=[pltpu.VMEM((1, EP), jnp.float32)],
    )(eids3)


def _meta2_body(eids_ref, ranks_ref, offs_ref, posa_ref, posb_ref):
    eids = eids_ref[0]
    lane = jax.lax.broadcasted_iota(jnp.int32, (EBLK, EP), 1)
    onehot = eids == lane
    offsel = jnp.sum(jnp.where(onehot, offs_ref[...], 0), axis=1,
                     keepdims=True)
    pos = ranks_ref[0] + offsel
    posa_ref[...] = pos.reshape(1, EBLK, 1)
    posb_ref[...] = (pos + NROWS).reshape(1, EBLK, 1)


def _meta2(eids3, ranks3, offs):
    return pl.pallas_call(
        _meta2_body,
        grid=(NEB,),
        in_specs=[
            pl.BlockSpec((1, EBLK, 1), lambda s: (s, 0, 0)),
            pl.BlockSpec((1, EBLK, 1), lambda s: (s, 0, 0)),
            pl.BlockSpec((1, EP), lambda s: (0, 0)),
        ],
        out_specs=[
            pl.BlockSpec((1, EBLK, 1), lambda s: (s, 0, 0)),
            pl.BlockSpec((1, EBLK, 1), lambda s: (s, 0, 0)),
        ],
        out_shape=[
            jax.ShapeDtypeStruct((NEB, EBLK, 1), jnp.int32),
            jax.ShapeDtypeStruct((NEB, EBLK, 1), jnp.int32),
        ],
    )(eids3, ranks3, offs)


_NC = 2
_NW = 32
_TPW = B // _NW
_CH = 16
_NCH = _TPW // _CH


def _dispatch(x, pos0, pos1):
    mesh = plsc.VectorSubcoreMesh(core_axis_name="c", subcore_axis_name="s")

    @functools.partial(
        pl.kernel,
        mesh=mesh,
        out_type=jax.ShapeDtypeStruct((NROWS, D_IN), jnp.float32),
        scratch_types=[
            pltpu.VMEM((_CH, D_IN), jnp.float32),
            pltpu.VMEM((_CH,), jnp.int32),
            pltpu.VMEM((_CH,), jnp.int32),
            pltpu.SemaphoreType.DMA,
            pltpu.SemaphoreType.DMA,
        ],
    )
    def k(x_hbm, pos0_hbm, pos1_hbm, xs_hbm, rows_v, idx0_v, idx1_v, s0, s1):
        wid = lax.axis_index("s") * _NC + lax.axis_index("c")
        for c in range(_NCH):
            tok0 = wid * _TPW + c * _CH
            pltpu.sync_copy(x_hbm.at[pl.ds(tok0, _CH)], rows_v)
            pltpu.sync_copy(pos0_hbm.at[pl.ds(tok0, _CH)], idx0_v)
            pltpu.sync_copy(pos1_hbm.at[pl.ds(tok0, _CH)], idx1_v)
            cp0 = pltpu.async_copy(rows_v, xs_hbm.at[idx0_v], s0)
            cp1 = pltpu.async_copy(rows_v, xs_hbm.at[idx1_v], s1)
            cp0.wait()
            cp1.wait()

    return k(x, pos0, pos1)


DK = 1024


def _ffn_body(m_ref, xs_ref, w1_ref, b1_ref, w2_ref, b2_ref, y_ref, h_ref):
    hb = pl.program_id(0)
    jt = pl.program_id(1)
    kb = pl.program_id(2)
    nact = m_ref[EP - 1]

    @pl.when(jt < nact)
    def _():
        @pl.when(kb < 2)
        def _():
            part = jnp.dot(xs_ref[...].astype(jnp.bfloat16),
                           w1_ref[0].astype(jnp.bfloat16),
                           preferred_element_type=jnp.float32)

            @pl.when(kb == 0)
            def _():
                h_ref[...] = part

            @pl.when(kb == 1)
            def _():
                h_ref[...] = jnp.maximum(h_ref[...] + part + b1_ref[0], 0.0)

        @pl.when(kb >= 2)
        def _():
            hs = h_ref[:, pl.ds((kb - 2) * DK, DK)]
            part2 = jnp.dot(hs.astype(jnp.bfloat16),
                            w2_ref[0].astype(jnp.bfloat16),
                            preferred_element_type=jnp.float32)

            @pl.when(kb == 2)
            def _():
                y_ref[...] = part2 + jnp.where(hb == 0, b2_ref[0], 0.0)

            @pl.when(kb == 3)
            def _():
                y_ref[...] += part2


def _ffn(tmeta, xs, w1f, b1r, w2f, b2r):
    mn = lambda kb: jnp.minimum(kb, 1)
    mx = lambda kb: jnp.maximum(kb - 2, 0)
    grid_spec = pltpu.PrefetchScalarGridSpec(
        num_scalar_prefetch=1,
        grid=(HB, NT, 4),
        in_specs=[
            pl.BlockSpec((T, DK), lambda hb, jt, kb, m: (jt, mn(kb))),
            pl.BlockSpec((1, DK, DH), lambda hb, jt, kb, m: (m[jt], mn(kb), hb)),
            pl.BlockSpec((1, 1, DH), lambda hb, jt, kb, m: (m[jt], 0, hb)),
            pl.BlockSpec((1, DK, D_OUT),
                         lambda hb, jt, kb, m: (m[jt], hb * 2 + mx(kb), 0)),
            pl.BlockSpec((1, 1, D_OUT), lambda hb, jt, kb, m: (m[jt], 0, 0)),
        ],
        out_specs=pl.BlockSpec((T, D_OUT),
                               lambda hb, jt, kb, m: (hb * NT + jt, 0)),
        scratch_shapes=[pltpu.VMEM((T, DH), jnp.float32)],
    )
    return pl.pallas_call(
        _ffn_body,
        grid_spec=grid_spec,
        out_shape=jax.ShapeDtypeStruct((HB * NROWS, D_OUT), jnp.float32),
        compiler_params=pltpu.CompilerParams(
            dimension_semantics=("arbitrary", "arbitrary", "arbitrary"),
        ),
    )(tmeta, xs, w1f, b1r, w2f, b2r)


_CHE = 8
_NCHE = _TPW // _CHE


def _combine(yflat, p0a, p0b, p1a, p1b, w0, w1):
    mesh = plsc.VectorSubcoreMesh(core_axis_name="c", subcore_axis_name="s")

    @functools.partial(
        pl.kernel,
        mesh=mesh,
        out_type=jax.ShapeDtypeStruct((B, D_OUT), jnp.float32),
        scratch_types=[
            pltpu.VMEM((_CHE, D_OUT), jnp.float32),
            pltpu.VMEM((_CHE, D_OUT), jnp.float32),
            pltpu.VMEM((_CHE, D_OUT), jnp.float32),
            pltpu.VMEM((_CHE, D_OUT), jnp.float32),
            pltpu.VMEM((_CHE, D_OUT), jnp.float32),
            pltpu.VMEM((_CHE,), jnp.int32),
            pltpu.VMEM((_CHE,), jnp.int32),
            pltpu.VMEM((_CHE,), jnp.int32),
            pltpu.VMEM((_CHE,), jnp.int32),
            pltpu.VMEM((16,), jnp.float32),
            pltpu.VMEM((16,), jnp.float32),
            pltpu.SemaphoreType.DMA,
            pltpu.SemaphoreType.DMA,
            pltpu.SemaphoreType.DMA,
            pltpu.SemaphoreType.DMA,
        ],
    )
    def k(y_hbm, p0a_hbm, p0b_hbm, p1a_hbm, p1b_hbm, w0_hbm, w1_hbm, out_hbm,
          y0a_v, y0b_v, y1a_v, y1b_v, out_v, i0a_v, i0b_v, i1a_v, i1b_v,
          w0_v, w1_v, s0, s1, s2, s3):
        wid = lax.axis_index("s") * _NC + lax.axis_index("c")
        for c in range(_NCHE):
            tok0 = wid * _TPW + c * _CHE
            sl = pl.ds(tok0, _CHE)
            pltpu.sync_copy(p0a_hbm.at[sl], i0a_v)
            pltpu.sync_copy(p0b_hbm.at[sl], i0b_v)
            pltpu.sync_copy(p1a_hbm.at[sl], i1a_v)
            pltpu.sync_copy(p1b_hbm.at[sl], i1b_v)
            if c % 2 == 0:
                slw = pl.ds(wid * _TPW + (c // 2) * 16, 16)
                pltpu.sync_copy(w0_hbm.at[slw], w0_v)
                pltpu.sync_copy(w1_hbm.at[slw], w1_v)
            c0 = pltpu.async_copy(y_hbm.at[i0a_v], y0a_v, s0)
            c1 = pltpu.async_copy(y_hbm.at[i0b_v], y0b_v, s1)
            c2 = pltpu.async_copy(y_hbm.at[i1a_v], y1a_v, s2)
            c3 = pltpu.async_copy(y_hbm.at[i1b_v], y1b_v, s3)
            c0.wait()
            c1.wait()
            c2.wait()
            c3.wait()
            w0reg = w0_v[...]
            w1reg = w1_v[...]
            for j in range(_CHE):
                jj = (c % 2) * _CHE + j
                bw0 = jnp.full((16,), w0reg[jj], jnp.float32)
                bw1 = jnp.full((16,), w1reg[jj], jnp.float32)

                def body(kk, carry):
                    ds = pl.ds(kk * 16, 16)
                    a = y0a_v[j, ds] + y0b_v[j, ds]
                    bb = y1a_v[j, ds] + y1b_v[j, ds]
                    out_v[j, ds] = a * bw0 + bb * bw1
                    return carry

                lax.fori_loop(0, D_OUT // 16, body, 0)
            pltpu.sync_copy(out_v, out_hbm.at[sl])

    return k(yflat, p0a, p0b, p1a, p1b, w0, w1)


@jax.jit
def kernel(x, W1, b1, W2, b2, Wg, bg):
    wgp = jnp.zeros((D_IN, EP), jnp.float32).at[:, :E].set(Wg)
    bgp = jnp.full((1, EP), NEG, jnp.float32).at[0, :E].set(bg)
    e1, e2, w0, w1, loss = _gating(x, wgp, bgp)

    eids = jnp.concatenate([e1.reshape(B), e2.reshape(B)])
    eids3 = eids.reshape(NEB, EBLK, 1)
    ranks3, offs, tmeta = _meta1(eids3)
    posa3, posb3 = _meta2(eids3, ranks3, offs)

    posa = posa3.reshape(NE)
    posb = posb3.reshape(NE)
    p0a, p1a = posa[:B], posa[B:]
    p0b, p1b = posb[:B], posb[B:]

    xs = _dispatch(x, p0a, p1a)

    yflat = _ffn(tmeta.reshape(EP), xs, W1, b1[:, None, :], W2,
                 b2[:, None, :])

    out = _combine(yflat, p0a, p0b, p1a, p1b, w0.reshape(B), w1.reshape(B))
    return out, loss[0, 0]


# speedup vs baseline: 1.3411x; 1.3411x over previous
import functools

import jax
import jax.numpy as jnp
from jax import lax
from jax.experimental import pallas as pl
from jax.experimental.pallas import tpu as pltpu
from jax.experimental.pallas import tpu_sc as plsc

E = 8
TOP_K = 2
D_IN = 2048
D_HID = 4096
D_OUT = 2048
B = 4096

EP = 128
BT = 256
NBG = B // BT
NEG = -1e30

T = 256
NROWS = 2 * B + E * T
NT = NROWS // T
NE = 2 * B
EBLK = 512
NEB = NE // EBLK

HB = 2
DH = D_HID // HB


def _gating_body(x_ref, wg_ref, bg_ref, e1_ref, e2_ref, w0_ref, w1_ref,
                 loss_ref, usage_ref):
    jb = pl.program_id(0)
    logits = jax.lax.dot(x_ref[...], wg_ref[...],
                         precision=jax.lax.Precision.DEFAULT) + bg_ref[...]
    iota = jax.lax.broadcasted_iota(jnp.int32, (BT, EP), 1)
    big = jnp.int32(2**30)
    m1 = jnp.max(logits, axis=1, keepdims=True)
    i1 = jnp.min(jnp.where(logits == m1, iota, big), axis=1, keepdims=True)
    l2 = jnp.where(iota == i1, NEG, logits)
    m2 = jnp.max(l2, axis=1, keepdims=True)
    i2 = jnp.min(jnp.where(l2 == m2, iota, big), axis=1, keepdims=True)
    d = jnp.exp(m2 - m1)
    w0 = 1.0 / (1.0 + d)
    w1 = d / (1.0 + d)
    e1_ref[...] = i1.reshape(1, BT, 1)
    e2_ref[...] = i2.reshape(1, BT, 1)
    w0_ref[...] = w0.reshape(1, BT, 1)
    w1_ref[...] = w1.reshape(1, BT, 1)
    g = jnp.exp(logits - m1)
    g = g / jnp.sum(g, axis=1, keepdims=True)
    gsum = jnp.sum(g, axis=0, keepdims=True)

    @pl.when(jb == 0)
    def _():
        usage_ref[...] = jnp.zeros_like(usage_ref)

    usage_ref[...] += gsum

    @pl.when(jb == NBG - 1)
    def _():
        usage = usage_ref[...] / B
        loss_ref[...] = jnp.reshape(E * jnp.sum(usage * usage), (1, 1))


def _gating(x, wgp, bgp):
    return pl.pallas_call(
        _gating_body,
        grid=(NBG,),
        in_specs=[
            pl.BlockSpec((BT, D_IN), lambda jb: (jb, 0)),
            pl.BlockSpec((D_IN, EP), lambda jb: (0, 0)),
            pl.BlockSpec((1, EP), lambda jb: (0, 0)),
        ],
        out_specs=[
            pl.BlockSpec((1, BT, 1), lambda jb: (jb, 0, 0)),
            pl.BlockSpec((1, BT, 1), lambda jb: (jb, 0, 0)),
            pl.BlockSpec((1, BT, 1), lambda jb: (jb, 0, 0)),
            pl.BlockSpec((1, BT, 1), lambda jb: (jb, 0, 0)),
            pl.BlockSpec((1, 1), lambda jb: (0, 0)),
        ],
        out_shape=[
            jax.ShapeDtypeStruct((NBG, BT, 1), jnp.int32),
            jax.ShapeDtypeStruct((NBG, BT, 1), jnp.int32),
            jax.ShapeDtypeStruct((NBG, BT, 1), jnp.float32),
            jax.ShapeDtypeStruct((NBG, BT, 1), jnp.float32),
            jax.ShapeDtypeStruct((1, 1), jnp.float32),
        ],
        scratch_shapes=[pltpu.VMEM((1, EP), jnp.float32)],
    )(x, wgp, bgp)


def _meta1_body(eids_ref, ranks_ref, offs_ref, tmeta_ref, carry_ref):
    s = pl.program_id(0)
    eids = eids_ref[0]
    lane = jax.lax.broadcasted_iota(jnp.int32, (EBLK, EP), 1)
    onehot = (eids == lane).astype(jnp.float32)
    r0 = jax.lax.broadcasted_iota(jnp.int32, (EBLK, EBLK), 0)
    r1 = jax.lax.broadcasted_iota(jnp.int32, (EBLK, EBLK), 1)
    lstrict = (r0 > r1).astype(jnp.float32)
    run = jax.lax.dot(lstrict, onehot,
                      precision=jax.lax.Precision.DEFAULT)

    @pl.when(s == 0)
    def _():
        carry_ref[...] = jnp.zeros_like(carry_ref)

    carry = carry_ref[...]
    rank = jnp.sum((run + carry) * onehot, axis=1, keepdims=True)
    ranks_ref[...] = rank.astype(jnp.int32).reshape(1, EBLK, 1)
    carry_ref[...] = carry + jnp.sum(onehot, axis=0, keepdims=True)

    @pl.when(s == NEB - 1)
    def _():
        counts = carry_ref[...]
        padded = jnp.ceil(counts / T) * T
        up0 = jax.lax.broadcasted_iota(jnp.int32, (EP, EP), 0)
        up1 = jax.lax.broadcasted_iota(jnp.int32, (EP, EP), 1)
        umask = (up0 < up1).astype(jnp.float32)
        offs = jax.lax.dot(padded, umask,
                           precision=jax.lax.Precision.HIGHEST)
        offs_ref[...] = offs.astype(jnp.int32)
        ends = (offs + padded) / T
        jt0 = jax.lax.broadcasted_iota(jnp.int32, (EP, EP), 0).astype(jnp.float32)
        laner = jax.lax.broadcasted_iota(jnp.int32, (EP, EP), 1)
        ge = jnp.where((laner < E) & (jt0 >= jnp.broadcast_to(ends, (EP, EP))),
                       1.0, 0.0)
        te = jnp.sum(ge, axis=1, keepdims=True)
        te = jnp.minimum(te, E - 1.0)
        nact = jnp.sum(padded) / T
        col = jax.lax.broadcasted_iota(jnp.int32, (EP, 1), 0)
        tmeta_ref[...] = jnp.where(col == EP - 1, nact, te).astype(jnp.int32)


def _meta1(eids3):
    return pl.pallas_call(
        _meta1_body,
        grid=(NEB,),
        in_specs=[pl.BlockSpec((1, EBLK, 1), lambda s: (s, 0, 0))],
        out_specs=[
            pl.BlockSpec((1, EBLK, 1), lambda s: (s, 0, 0)),
            pl.BlockSpec((1, EP), lambda s: (0, 0)),
            pl.BlockSpec((EP, 1), lambda s: (0, 0)),
        ],
        out_shape=[
            jax.ShapeDtypeStruct((NEB, EBLK, 1), jnp.int32),
            jax.ShapeDtypeStruct((1, EP), jnp.int32),
            jax.ShapeDtypeStruct((EP, 1), jnp.int32),
        ],
        scratch_shapes=[pltpu.VMEM((1, EP), jnp.float32)],
    )(eids3)


def _meta2_body(eids_ref, ranks_ref, offs_ref, posa_ref, posb_ref):
    eids = eids_ref[0]
    lane = jax.lax.broadcasted_iota(jnp.int32, (EBLK, EP), 1)
    onehot = eids == lane
    offsel = jnp.sum(jnp.where(onehot, offs_ref[...], 0), axis=1,
                     keepdims=True)
    pos = ranks_ref[0] + offsel
    posa_ref[...] = pos.reshape(1, EBLK, 1)
    posb_ref[...] = (pos + NROWS).reshape(1, EBLK, 1)


def _meta2(eids3, ranks3, offs):
    return pl.pallas_call(
        _meta2_body,
        grid=(NEB,),
        in_specs=[
            pl.BlockSpec((1, EBLK, 1), lambda s: (s, 0, 0)),
            pl.BlockSpec((1, EBLK, 1), lambda s: (s, 0, 0)),
            pl.BlockSpec((1, EP), lambda s: (0, 0)),
        ],
        out_specs=[
            pl.BlockSpec((1, EBLK, 1), lambda s: (s, 0, 0)),
            pl.BlockSpec((1, EBLK, 1), lambda s: (s, 0, 0)),
        ],
        out_shape=[
            jax.ShapeDtypeStruct((NEB, EBLK, 1), jnp.int32),
            jax.ShapeDtypeStruct((NEB, EBLK, 1), jnp.int32),
        ],
    )(eids3, ranks3, offs)


_NC = 2
_NW = 32
_TPW = B // _NW
_CH = 16
_NCH = _TPW // _CH


def _dispatch(x, pos0, pos1):
    mesh = plsc.VectorSubcoreMesh(core_axis_name="c", subcore_axis_name="s")

    @functools.partial(
        pl.kernel,
        mesh=mesh,
        out_type=jax.ShapeDtypeStruct((NROWS, D_IN), jnp.float32),
        scratch_types=[
            pltpu.VMEM((_CH, D_IN), jnp.float32),
            pltpu.VMEM((_CH, D_IN), jnp.float32),
            pltpu.VMEM((_CH,), jnp.int32),
            pltpu.VMEM((_CH,), jnp.int32),
            pltpu.VMEM((_CH,), jnp.int32),
            pltpu.VMEM((_CH,), jnp.int32),
            pltpu.SemaphoreType.DMA,
            pltpu.SemaphoreType.DMA,
            pltpu.SemaphoreType.DMA,
            pltpu.SemaphoreType.DMA,
        ],
    )
    def k(x_hbm, pos0_hbm, pos1_hbm, xs_hbm, rows_a, rows_b,
          i0a, i1a, i0b, i1b, s0a, s1a, s0b, s1b):
        wid = lax.axis_index("s") * _NC + lax.axis_index("c")
        rows = (rows_a, rows_b)
        idx0 = (i0a, i0b)
        idx1 = (i1a, i1b)
        sems = ((s0a, s1a), (s0b, s1b))
        pend = [None, None]
        for c in range(_NCH):
            bsel = c % 2
            if pend[bsel] is not None:
                pend[bsel][0].wait()
                pend[bsel][1].wait()
            tok0 = wid * _TPW + c * _CH
            pltpu.sync_copy(x_hbm.at[pl.ds(tok0, _CH)], rows[bsel])
            pltpu.sync_copy(pos0_hbm.at[pl.ds(tok0, _CH)], idx0[bsel])
            pltpu.sync_copy(pos1_hbm.at[pl.ds(tok0, _CH)], idx1[bsel])
            pend[bsel] = (
                pltpu.async_copy(rows[bsel], xs_hbm.at[idx0[bsel]], sems[bsel][0]),
                pltpu.async_copy(rows[bsel], xs_hbm.at[idx1[bsel]], sems[bsel][1]),
            )
        for p in pend:
            if p is not None:
                p[0].wait()
                p[1].wait()

    return k(x, pos0, pos1)


def _ffn_body(m_ref, xs_ref, w1_ref, b1_ref, w2_ref, b2_ref, y_ref):
    hb = pl.program_id(0)
    jt = pl.program_id(1)
    nact = m_ref[EP - 1]

    @pl.when(jt < nact)
    def _():
        xb = xs_ref[...].astype(jnp.bfloat16)
        h = jnp.maximum(
            jnp.dot(xb, w1_ref[0], preferred_element_type=jnp.float32)
            + b1_ref[0], 0.0)
        y = jnp.dot(h.astype(jnp.bfloat16), w2_ref[0],
                    preferred_element_type=jnp.float32)
        y_ref[...] = y + jnp.where(hb == 0, b2_ref[0], 0.0)


def _ffn(tmeta, xs, w1b, b1r, w2b, b2r):
    grid_spec = pltpu.PrefetchScalarGridSpec(
        num_scalar_prefetch=1,
        grid=(HB, NT),
        in_specs=[
            pl.BlockSpec((T, D_IN), lambda hb, jt, m: (jt, 0)),
            pl.BlockSpec((1, D_IN, DH), lambda hb, jt, m: (m[jt], 0, hb)),
            pl.BlockSpec((1, 1, DH), lambda hb, jt, m: (m[jt], 0, hb)),
            pl.BlockSpec((1, DH, D_OUT), lambda hb, jt, m: (m[jt], hb, 0)),
            pl.BlockSpec((1, 1, D_OUT), lambda hb, jt, m: (m[jt], 0, 0)),
        ],
        out_specs=pl.BlockSpec((T, D_OUT), lambda hb, jt, m: (hb * NT + jt, 0)),
    )
    return pl.pallas_call(
        _ffn_body,
        grid_spec=grid_spec,
        out_shape=jax.ShapeDtypeStruct((HB * NROWS, D_OUT), jnp.float32),
        compiler_params=pltpu.CompilerParams(
            dimension_semantics=("arbitrary", "arbitrary"),
        ),
    )(tmeta, xs, w1b, b1r, w2b, b2r)


_CHE = 8
_NCHE = _TPW // _CHE


def _combine(yflat, p0a, p0b, p1a, p1b, w0, w1):
    mesh = plsc.VectorSubcoreMesh(core_axis_name="c", subcore_axis_name="s")

    @functools.partial(
        pl.kernel,
        mesh=mesh,
        out_type=jax.ShapeDtypeStruct((B, D_OUT), jnp.float32),
        scratch_types=[
            pltpu.VMEM((2 * _CHE, D_OUT), jnp.float32),
            pltpu.VMEM((2 * _CHE, D_OUT), jnp.float32),
            pltpu.VMEM((_CHE, D_OUT), jnp.float32),
            pltpu.VMEM((2 * _CHE,), jnp.int32),
            pltpu.VMEM((2 * _CHE,), jnp.int32),
            pltpu.VMEM((16,), jnp.float32),
            pltpu.VMEM((16,), jnp.float32),
            pltpu.SemaphoreType.DMA,
            pltpu.SemaphoreType.DMA,
        ],
    )
    def k(y_hbm, p0a_hbm, p0b_hbm, p1a_hbm, p1b_hbm, w0_hbm, w1_hbm, out_hbm,
          yb0_v, yb1_v, out_v, i0_v, i1_v, w0_v, w1_v, s0, s1):
        wid = lax.axis_index("s") * _NC + lax.axis_index("c")
        for c in range(_NCHE):
            tok0 = wid * _TPW + c * _CHE
            sl = pl.ds(tok0, _CHE)
            lo = pl.ds(0, _CHE)
            hi = pl.ds(_CHE, _CHE)
            pltpu.sync_copy(p0a_hbm.at[sl], i0_v.at[lo])
            pltpu.sync_copy(p0b_hbm.at[sl], i0_v.at[hi])
            pltpu.sync_copy(p1a_hbm.at[sl], i1_v.at[lo])
            pltpu.sync_copy(p1b_hbm.at[sl], i1_v.at[hi])
            c0 = pltpu.async_copy(y_hbm.at[i0_v], yb0_v, s0)
            c1 = pltpu.async_copy(y_hbm.at[i1_v], yb1_v, s1)
            if c % 2 == 0:
                slw = pl.ds(wid * _TPW + (c // 2) * 16, 16)
                pltpu.sync_copy(w0_hbm.at[slw], w0_v)
                pltpu.sync_copy(w1_hbm.at[slw], w1_v)
            w0reg = w0_v[...]
            w1reg = w1_v[...]
            c0.wait()
            for j in range(_CHE):
                jj = (c % 2) * _CHE + j
                bw0 = jnp.full((16,), w0reg[jj], jnp.float32)

                def body0(kk, carry):
                    ds = pl.ds(kk * 16, 16)
                    out_v[j, ds] = (yb0_v[j, ds] + yb0_v[j + _CHE, ds]) * bw0
                    return carry

                lax.fori_loop(0, D_OUT // 16, body0, 0)
            c1.wait()
            for j in range(_CHE):
                jj = (c % 2) * _CHE + j
                bw1 = jnp.full((16,), w1reg[jj], jnp.float32)

                def body1(kk, carry):
                    ds = pl.ds(kk * 16, 16)
                    out_v[j, ds] += (yb1_v[j, ds] + yb1_v[j + _CHE, ds]) * bw1
                    return carry

                lax.fori_loop(0, D_OUT // 16, body1, 0)
            pltpu.sync_copy(out_v, out_hbm.at[sl])

    return k(yflat, p0a, p0b, p1a, p1b, w0, w1)


@jax.jit
def kernel(x, W1, b1, W2, b2, Wg, bg):
    wgp = jnp.zeros((D_IN, EP), jnp.float32).at[:, :E].set(Wg)
    bgp = jnp.full((1, EP), NEG, jnp.float32).at[0, :E].set(bg)
    e1, e2, w0, w1, loss = _gating(x, wgp, bgp)

    eids = jnp.concatenate([e1.reshape(B), e2.reshape(B)])
    eids3 = eids.reshape(NEB, EBLK, 1)
    ranks3, offs, tmeta = _meta1(eids3)
    posa3, posb3 = _meta2(eids3, ranks3, offs)

    posa = posa3.reshape(NE)
    posb = posb3.reshape(NE)
    p0a, p1a = posa[:B], posa[B:]
    p0b, p1b = posb[:B], posb[B:]

    xs = _dispatch(x, p0a, p1a)

    yflat = _ffn(tmeta.reshape(EP), xs, W1.astype(jnp.bfloat16),
                 b1[:, None, :], W2.astype(jnp.bfloat16), b2[:, None, :])

    out = _combine(yflat, p0a, p0b, p1a, p1b, w0.reshape(B), w1.reshape(B))
    return out, loss[0, 0]


# speedup vs baseline: 1.4584x; 1.0875x over previous
import functools

import jax
import jax.numpy as jnp
from jax import lax
from jax.experimental import pallas as pl
from jax.experimental.pallas import tpu as pltpu
from jax.experimental.pallas import tpu_sc as plsc

E = 8
TOP_K = 2
D_IN = 2048
D_HID = 4096
D_OUT = 2048
B = 4096

EP = 128
BT = 256
NBG = B // BT
NEG = -1e30

T = 256
NROWS = 2 * B + E * T
NT = NROWS // T
NE = 2 * B
EBLK = 512
NEB = NE // EBLK

HB = 2
DH = D_HID // HB


def _gating_body(x_ref, wg_ref, bg_ref, e1_ref, e2_ref, w0_ref, w1_ref,
                 loss_ref, usage_ref):
    jb = pl.program_id(0)
    logits = jax.lax.dot(x_ref[...], wg_ref[...],
                         precision=jax.lax.Precision.DEFAULT) + bg_ref[...]
    iota = jax.lax.broadcasted_iota(jnp.int32, (BT, EP), 1)
    big = jnp.int32(2**30)
    m1 = jnp.max(logits, axis=1, keepdims=True)
    i1 = jnp.min(jnp.where(logits == m1, iota, big), axis=1, keepdims=True)
    l2 = jnp.where(iota == i1, NEG, logits)
    m2 = jnp.max(l2, axis=1, keepdims=True)
    i2 = jnp.min(jnp.where(l2 == m2, iota, big), axis=1, keepdims=True)
    d = jnp.exp(m2 - m1)
    w0 = 1.0 / (1.0 + d)
    w1 = d / (1.0 + d)
    e1_ref[...] = i1.reshape(1, BT, 1)
    e2_ref[...] = i2.reshape(1, BT, 1)
    w0_ref[...] = w0.reshape(1, BT, 1)
    w1_ref[...] = w1.reshape(1, BT, 1)
    g = jnp.exp(logits - m1)
    g = g / jnp.sum(g, axis=1, keepdims=True)
    gsum = jnp.sum(g, axis=0, keepdims=True)

    @pl.when(jb == 0)
    def _():
        usage_ref[...] = jnp.zeros_like(usage_ref)

    usage_ref[...] += gsum

    @pl.when(jb == NBG - 1)
    def _():
        usage = usage_ref[...] / B
        loss_ref[...] = jnp.reshape(E * jnp.sum(usage * usage), (1, 1))


def _gating(x, wgp, bgp):
    return pl.pallas_call(
        _gating_body,
        grid=(NBG,),
        in_specs=[
            pl.BlockSpec((BT, D_IN), lambda jb: (jb, 0)),
            pl.BlockSpec((D_IN, EP), lambda jb: (0, 0)),
            pl.BlockSpec((1, EP), lambda jb: (0, 0)),
        ],
        out_specs=[
            pl.BlockSpec((1, BT, 1), lambda jb: (jb, 0, 0)),
            pl.BlockSpec((1, BT, 1), lambda jb: (jb, 0, 0)),
            pl.BlockSpec((1, BT, 1), lambda jb: (jb, 0, 0)),
            pl.BlockSpec((1, BT, 1), lambda jb: (jb, 0, 0)),
            pl.BlockSpec((1, 1), lambda jb: (0, 0)),
        ],
        out_shape=[
            jax.ShapeDtypeStruct((NBG, BT, 1), jnp.int32),
            jax.ShapeDtypeStruct((NBG, BT, 1), jnp.int32),
            jax.ShapeDtypeStruct((NBG, BT, 1), jnp.float32),
            jax.ShapeDtypeStruct((NBG, BT, 1), jnp.float32),
            jax.ShapeDtypeStruct((1, 1), jnp.float32),
        ],
        scratch_shapes=[pltpu.VMEM((1, EP), jnp.float32)],
    )(x, wgp, bgp)


def _meta1_body(eids_ref, ranks_ref, offs_ref, tmeta_ref, carry_ref):
    s = pl.program_id(0)
    eids = eids_ref[0]
    lane = jax.lax.broadcasted_iota(jnp.int32, (EBLK, EP), 1)
    onehot = (eids == lane).astype(jnp.float32)
    r0 = jax.lax.broadcasted_iota(jnp.int32, (EBLK, EBLK), 0)
    r1 = jax.lax.broadcasted_iota(jnp.int32, (EBLK, EBLK), 1)
    lstrict = (r0 > r1).astype(jnp.float32)
    run = jax.lax.dot(lstrict, onehot,
                      precision=jax.lax.Precision.DEFAULT)

    @pl.when(s == 0)
    def _():
        carry_ref[...] = jnp.zeros_like(carry_ref)

    carry = carry_ref[...]
    rank = jnp.sum((run + carry) * onehot, axis=1, keepdims=True)
    ranks_ref[...] = rank.astype(jnp.int32).reshape(1, EBLK, 1)
    carry_ref[...] = carry + jnp.sum(onehot, axis=0, keepdims=True)

    @pl.when(s == NEB - 1)
    def _():
        counts = carry_ref[...]
        padded = jnp.ceil(counts / T) * T
        up0 = jax.lax.broadcasted_iota(jnp.int32, (EP, EP), 0)
        up1 = jax.lax.broadcasted_iota(jnp.int32, (EP, EP), 1)
        umask = (up0 < up1).astype(jnp.float32)
        offs = jax.lax.dot(padded, umask,
                           precision=jax.lax.Precision.HIGHEST)
        offs_ref[...] = offs.astype(jnp.int32)
        ends = (offs + padded) / T
        jt0 = jax.lax.broadcasted_iota(jnp.int32, (EP, EP), 0).astype(jnp.float32)
        laner = jax.lax.broadcasted_iota(jnp.int32, (EP, EP), 1)
        ge = jnp.where((laner < E) & (jt0 >= jnp.broadcast_to(ends, (EP, EP))),
                       1.0, 0.0)
        te = jnp.sum(ge, axis=1, keepdims=True)
        te = jnp.minimum(te, E - 1.0)
        nact = jnp.sum(padded) / T
        col = jax.lax.broadcasted_iota(jnp.int32, (EP, 1), 0)
        tmeta_ref[...] = jnp.where(col == EP - 1, nact, te).astype(jnp.int32)


def _meta1(eids3):
    return pl.pallas_call(
        _meta1_body,
        grid=(NEB,),
        in_specs=[pl.BlockSpec((1, EBLK, 1), lambda s: (s, 0, 0))],
        out_specs=[
            pl.BlockSpec((1, EBLK, 1), lambda s: (s, 0, 0)),
            pl.BlockSpec((1, EP), lambda s: (0, 0)),
            pl.BlockSpec((EP, 1), lambda s: (0, 0)),
        ],
        out_shape=[
            jax.ShapeDtypeStruct((NEB, EBLK, 1), jnp.int32),
            jax.ShapeDtypeStruct((1, EP), jnp.int32),
            jax.ShapeDtypeStruct((EP, 1), jnp.int32),
        ],
        scratch_shapes=[pltpu.VMEM((1, EP), jnp.float32)],
    )(eids3)


def _meta2_body(eids_ref, ranks_ref, offs_ref, posa_ref, posb_ref):
    eids = eids_ref[0]
    lane = jax.lax.broadcasted_iota(jnp.int32, (EBLK, EP), 1)
    onehot = eids == lane
    offsel = jnp.sum(jnp.where(onehot, offs_ref[...], 0), axis=1,
                     keepdims=True)
    pos = ranks_ref[0] + offsel
    posa_ref[...] = pos.reshape(1, EBLK, 1)
    posb_ref[...] = (pos + NROWS).reshape(1, EBLK, 1)


def _meta2(eids3, ranks3, offs):
    return pl.pallas_call(
        _meta2_body,
        grid=(NEB,),
        in_specs=[
            pl.BlockSpec((1, EBLK, 1), lambda s: (s, 0, 0)),
            pl.BlockSpec((1, EBLK, 1), lambda s: (s, 0, 0)),
            pl.BlockSpec((1, EP), lambda s: (0, 0)),
        ],
        out_specs=[
            pl.BlockSpec((1, EBLK, 1), lambda s: (s, 0, 0)),
            pl.BlockSpec((1, EBLK, 1), lambda s: (s, 0, 0)),
        ],
        out_shape=[
            jax.ShapeDtypeStruct((NEB, EBLK, 1), jnp.int32),
            jax.ShapeDtypeStruct((NEB, EBLK, 1), jnp.int32),
        ],
    )(eids3, ranks3, offs)


_NC = 2
_NW = 32
_TPW = B // _NW
_CH = 16
_NCH = _TPW // _CH


def _dispatch(x, pos0, pos1):
    mesh = plsc.VectorSubcoreMesh(core_axis_name="c", subcore_axis_name="s")

    @functools.partial(
        pl.kernel,
        mesh=mesh,
        out_type=jax.ShapeDtypeStruct((NROWS, D_IN), jnp.float32),
        scratch_types=[
            pltpu.VMEM((_CH, D_IN), jnp.float32),
            pltpu.VMEM((_CH, D_IN), jnp.float32),
            pltpu.VMEM((_CH,), jnp.int32),
            pltpu.VMEM((_CH,), jnp.int32),
            pltpu.VMEM((_CH,), jnp.int32),
            pltpu.VMEM((_CH,), jnp.int32),
            pltpu.SemaphoreType.DMA,
            pltpu.SemaphoreType.DMA,
            pltpu.SemaphoreType.DMA,
            pltpu.SemaphoreType.DMA,
        ],
    )
    def k(x_hbm, pos0_hbm, pos1_hbm, xs_hbm, rows_a, rows_b,
          i0a, i1a, i0b, i1b, s0a, s1a, s0b, s1b):
        wid = lax.axis_index("s") * _NC + lax.axis_index("c")
        rows = (rows_a, rows_b)
        idx0 = (i0a, i0b)
        idx1 = (i1a, i1b)
        sems = ((s0a, s1a), (s0b, s1b))
        pend = [None, None]
        for c in range(_NCH):
            bsel = c % 2
            if pend[bsel] is not None:
                pend[bsel][0].wait()
                pend[bsel][1].wait()
            tok0 = wid * _TPW + c * _CH
            pltpu.sync_copy(x_hbm.at[pl.ds(tok0, _CH)], rows[bsel])
            pltpu.sync_copy(pos0_hbm.at[pl.ds(tok0, _CH)], idx0[bsel])
            pltpu.sync_copy(pos1_hbm.at[pl.ds(tok0, _CH)], idx1[bsel])
            pend[bsel] = (
                pltpu.async_copy(rows[bsel], xs_hbm.at[idx0[bsel]], sems[bsel][0]),
                pltpu.async_copy(rows[bsel], xs_hbm.at[idx1[bsel]], sems[bsel][1]),
            )
        for p in pend:
            if p is not None:
                p[0].wait()
                p[1].wait()

    return k(x, pos0, pos1)


def _ffn_body(m_ref, xs_ref, w1_ref, b1_ref, w2_ref, b2_ref, y_ref):
    hb = pl.program_id(0)
    jt = pl.program_id(1)
    nact = m_ref[EP - 1]

    @pl.when(jt < nact)
    def _():
        xb = xs_ref[...].astype(jnp.bfloat16)
        h = jnp.maximum(
            jnp.dot(xb, w1_ref[0], preferred_element_type=jnp.float32)
            + b1_ref[0], 0.0)
        y = jnp.dot(h.astype(jnp.bfloat16), w2_ref[0],
                    preferred_element_type=jnp.float32)
        y_ref[...] = y + jnp.where(hb == 0, b2_ref[0], 0.0)


def _ffn(tmeta, xs, w1b, b1r, w2b, b2r):
    grid_spec = pltpu.PrefetchScalarGridSpec(
        num_scalar_prefetch=1,
        grid=(HB, NT),
        in_specs=[
            pl.BlockSpec((T, D_IN), lambda hb, jt, m: (jt, 0)),
            pl.BlockSpec((1, D_IN, DH), lambda hb, jt, m: (m[jt], 0, hb)),
            pl.BlockSpec((1, 1, DH), lambda hb, jt, m: (m[jt], 0, hb)),
            pl.BlockSpec((1, DH, D_OUT), lambda hb, jt, m: (m[jt], hb, 0)),
            pl.BlockSpec((1, 1, D_OUT), lambda hb, jt, m: (m[jt], 0, 0)),
        ],
        out_specs=pl.BlockSpec((T, D_OUT), lambda hb, jt, m: (hb * NT + jt, 0)),
    )
    return pl.pallas_call(
        _ffn_body,
        grid_spec=grid_spec,
        out_shape=jax.ShapeDtypeStruct((HB * NROWS, D_OUT), jnp.float32),
        compiler_params=pltpu.CompilerParams(
            dimension_semantics=("arbitrary", "arbitrary"),
        ),
    )(tmeta, xs, w1b, b1r, w2b, b2r)


_CHE = 8
_NCHE = _TPW // _CHE


def _combine(yflat, p0a, p0b, p1a, p1b, w0, w1):
    mesh = plsc.VectorSubcoreMesh(core_axis_name="c", subcore_axis_name="s")

    @functools.partial(
        pl.kernel,
        mesh=mesh,
        out_type=jax.ShapeDtypeStruct((B, D_OUT), jnp.float32),
        scratch_types=[
            pltpu.VMEM((_CHE, D_OUT), jnp.float32),
            pltpu.VMEM((_CHE, D_OUT), jnp.float32),
            pltpu.VMEM((_CHE, D_OUT), jnp.float32),
            pltpu.VMEM((_CHE, D_OUT), jnp.float32),
            pltpu.VMEM((_CHE, D_OUT), jnp.float32),
            pltpu.VMEM((_CHE,), jnp.int32),
            pltpu.VMEM((_CHE,), jnp.int32),
            pltpu.VMEM((_CHE,), jnp.int32),
            pltpu.VMEM((_CHE,), jnp.int32),
            pltpu.VMEM((16,), jnp.float32),
            pltpu.VMEM((16,), jnp.float32),
            pltpu.SemaphoreType.DMA,
            pltpu.SemaphoreType.DMA,
            pltpu.SemaphoreType.DMA,
            pltpu.SemaphoreType.DMA,
        ],
    )
    def k(y_hbm, p0a_hbm, p0b_hbm, p1a_hbm, p1b_hbm, w0_hbm, w1_hbm, out_hbm,
          y0a_v, y0b_v, y1a_v, y1b_v, out_v, i0a_v, i0b_v, i1a_v, i1b_v,
          w0_v, w1_v, s0, s1, s2, s3):
        wid = lax.axis_index("s") * _NC + lax.axis_index("c")
        for c in range(_NCHE):
            tok0 = wid * _TPW + c * _CHE
            sl = pl.ds(tok0, _CHE)
            pltpu.sync_copy(p0a_hbm.at[sl], i0a_v)
            pltpu.sync_copy(p0b_hbm.at[sl], i0b_v)
            pltpu.sync_copy(p1a_hbm.at[sl], i1a_v)
            pltpu.sync_copy(p1b_hbm.at[sl], i1b_v)
            if c % 2 == 0:
                slw = pl.ds(wid * _TPW + (c // 2) * 16, 16)
                pltpu.sync_copy(w0_hbm.at[slw], w0_v)
                pltpu.sync_copy(w1_hbm.at[slw], w1_v)
            c0 = pltpu.async_copy(y_hbm.at[i0a_v], y0a_v, s0)
            c1 = pltpu.async_copy(y_hbm.at[i0b_v], y0b_v, s1)
            c2 = pltpu.async_copy(y_hbm.at[i1a_v], y1a_v, s2)
            c3 = pltpu.async_copy(y_hbm.at[i1b_v], y1b_v, s3)
            c0.wait()
            c1.wait()
            c2.wait()
            c3.wait()
            w0reg = w0_v[...]
            w1reg = w1_v[...]
            for j in range(_CHE):
                jj = (c % 2) * _CHE + j
                bw0 = jnp.full((16,), w0reg[jj], jnp.float32)
                bw1 = jnp.full((16,), w1reg[jj], jnp.float32)

                def body(kk, carry):
                    ds = pl.ds(kk * 16, 16)
                    a = y0a_v[j, ds] + y0b_v[j, ds]
                    bb = y1a_v[j, ds] + y1b_v[j, ds]
                    out_v[j, ds] = a * bw0 + bb * bw1
                    return carry

                lax.fori_loop(0, D_OUT // 16, body, 0)
            pltpu.sync_copy(out_v, out_hbm.at[sl])

    return k(yflat, p0a, p0b, p1a, p1b, w0, w1)


@jax.jit
def kernel(x, W1, b1, W2, b2, Wg, bg):
    wgp = jnp.zeros((D_IN, EP), jnp.float32).at[:, :E].set(Wg)
    bgp = jnp.full((1, EP), NEG, jnp.float32).at[0, :E].set(bg)
    e1, e2, w0, w1, loss = _gating(x, wgp, bgp)

    eids = jnp.concatenate([e1.reshape(B), e2.reshape(B)])
    eids3 = eids.reshape(NEB, EBLK, 1)
    ranks3, offs, tmeta = _meta1(eids3)
    posa3, posb3 = _meta2(eids3, ranks3, offs)

    posa = posa3.reshape(NE)
    posb = posb3.reshape(NE)
    p0a, p1a = posa[:B], posa[B:]
    p0b, p1b = posb[:B], posb[B:]

    xs = _dispatch(x, p0a, p1a)

    yflat = _ffn(tmeta.reshape(EP), xs, W1.astype(jnp.bfloat16),
                 b1[:, None, :], W2.astype(jnp.bfloat16), b2[:, None, :])

    out = _combine(yflat, p0a, p0b, p1a, p1b, w0.reshape(B), w1.reshape(B))
    return out, loss[0, 0]
